# baseline (speedup 1.0000x reference)
import jax
import jax.numpy as jnp
from jax.experimental import pallas as pl

NUM_CLASSES = 1000
BATCH_BLOCK = 256


def _onehot_block(x_ref, o_ref):
    idx = x_ref[...]
    cols = jax.lax.broadcasted_iota(jnp.int32, o_ref.shape, 2)
    o_ref[...] = (cols == idx[:, :, None]).astype(jnp.float32)


def kernel(x, table):
    del table
    B, S = x.shape
    nb = B // BATCH_BLOCK
    return pl.pallas_call(
        _onehot_block,
        grid=(nb,),
        in_specs=[pl.BlockSpec((BATCH_BLOCK, S), lambda i: (i, 0))],
        out_specs=pl.BlockSpec((BATCH_BLOCK, S, NUM_CLASSES), lambda i: (i, 0, 0)),
        out_shape=jax.ShapeDtypeStruct((B, S, NUM_CLASSES), jnp.float32),
    )(x)


# speedup vs baseline: 1.0107x; 1.0107x over previous
import jax
import jax.numpy as jnp
from jax.experimental import pallas as pl
from jax.experimental.pallas import tpu as pltpu

NUM_CLASSES = 1000
BATCH_BLOCK = 256


def _onehot_block(x_ref, o_ref):
    idx = x_ref[...]
    cols = jax.lax.broadcasted_iota(jnp.int32, o_ref.shape, 2)
    o_ref[...] = (cols == idx[:, :, None]).astype(jnp.float32)


def kernel(x, table):
    del table
    B, S = x.shape
    nb = B // BATCH_BLOCK
    return pl.pallas_call(
        _onehot_block,
        grid=(nb,),
        in_specs=[pl.BlockSpec((BATCH_BLOCK, S), lambda i: (i, 0))],
        out_specs=pl.BlockSpec((BATCH_BLOCK, S, NUM_CLASSES), lambda i: (i, 0, 0)),
        out_shape=jax.ShapeDtypeStruct((B, S, NUM_CLASSES), jnp.float32),
        compiler_params=pltpu.CompilerParams(
            dimension_semantics=("parallel",),
        ),
    )(x)
